# baseline (speedup 1.0000x reference)
import functools

import jax
import jax.numpy as jnp
from jax import lax
from jax.experimental import pallas as pl
from jax.experimental.pallas import tpu as pltpu, tpu_sc as plsc

N_HEADS = 4
N_EMB = 8192
HDIM = 16
N_TOK = 8192
TOK_BLK = 512
N_WORKERS = 32
IDX_CHUNK = 128


def _tc_body(x_ref, wt_ref, idx_ref):
    def mm(h):
        x_h = x_ref[:, h * HDIM:(h + 1) * HDIM]
        wt_h = wt_ref[h]
        s = lax.dot_general(x_h, wt_h, (((1,), (0,)), ((), ())),
                            preferred_element_type=jnp.float32)
        return s, wt_h

    def amax(s, wt_h, h):
        t = s - 0.5 * jnp.sum(wt_h * wt_h, axis=0)[None, :]
        return jnp.argmax(t, axis=1).astype(jnp.int32) + h * N_EMB

    idx_cols = []
    prev = mm(0)
    for h in range(1, N_HEADS):
        cur = mm(h)
        idx_cols.append(amax(*prev, h - 1))
        prev = cur
    idx_cols.append(amax(*prev, N_HEADS - 1))
    idx_ref[0] = jnp.stack(idx_cols, axis=1)


_B_PER_W = N_TOK * N_HEADS // N_WORKERS
_N_CHUNK = _B_PER_W // IDX_CHUNK


@functools.cache
def _make_sc_gather():
    @functools.partial(
        pl.kernel,
        out_type=(
            jax.ShapeDtypeStruct((N_TOK * N_HEADS, HDIM), jnp.float32),
            jax.ShapeDtypeStruct((N_WORKERS, HDIM), jnp.float32),
        ),
        mesh=plsc.VectorSubcoreMesh(core_axis_name="c", subcore_axis_name="s"),
        scratch_types=[
            pltpu.VMEM((_N_CHUNK, IDX_CHUNK), jnp.int32),
            pltpu.VMEM((_B_PER_W, HDIM), jnp.float32),
            pltpu.VMEM((_B_PER_W, HDIM), jnp.float32),
            pltpu.VMEM((HDIM,), jnp.float32),
            pltpu.SemaphoreType.DMA,
            pltpu.SemaphoreType.DMA,
        ],
        compiler_params=pltpu.CompilerParams(use_tc_tiling_on_sc=False),
    )
    def _sc_gather(idx_hbm, table_hbm, x_hbm, out_hbm, part_hbm,
                   idx_v, rows_v, x_v, part_v, sem, xsem):
        wid = lax.axis_index("s") * 2 + lax.axis_index("c")
        base = wid * _B_PER_W
        pltpu.sync_copy(idx_hbm.at[wid], idx_v)
        xcp = pltpu.async_copy(x_hbm.at[pl.ds(base, _B_PER_W)], x_v, xsem)
        copies = [
            pltpu.async_copy(table_hbm.at[idx_v.at[j]],
                             rows_v.at[pl.ds(j * IDX_CHUNK, IDX_CHUNK)], sem)
            for j in range(_N_CHUNK)
        ]
        for cp in copies:
            cp.wait()
        xcp.wait()

        def body(i, acc):
            d = rows_v[i] - x_v[i]
            return acc + d * d

        acc = lax.fori_loop(0, _B_PER_W, body, jnp.zeros((HDIM,), jnp.float32))
        part_v[...] = acc
        pltpu.sync_copy(part_v, part_hbm.at[wid])
        pltpu.sync_copy(rows_v, out_hbm.at[pl.ds(base, _B_PER_W)])

    return _sc_gather


def kernel(inputs, weights):
    x2d = inputs.reshape(N_TOK, N_HEADS * HDIM)
    wt = jnp.swapaxes(weights, 1, 2)

    n_tb = N_TOK // TOK_BLK
    gidx = pl.pallas_call(
        _tc_body,
        grid=(n_tb,),
        in_specs=[
            pl.BlockSpec((TOK_BLK, N_HEADS * HDIM), lambda t: (t, 0)),
            pl.BlockSpec((N_HEADS, HDIM, N_EMB), lambda t: (0, 0, 0)),
        ],
        out_specs=pl.BlockSpec((1, TOK_BLK, N_HEADS), lambda t: (t, 0, 0)),
        out_shape=jax.ShapeDtypeStruct((n_tb, TOK_BLK, N_HEADS), jnp.int32),
    )(x2d, wt)

    idx3 = gidx.reshape(N_WORKERS, _N_CHUNK, IDX_CHUNK)
    table = weights.reshape(N_HEADS * N_EMB, HDIM)
    xrows = inputs.reshape(N_TOK * N_HEADS, HDIM)
    q, parts = _make_sc_gather()(idx3, table, xrows)
    q = q.reshape(inputs.shape)

    loss = 0.25 * jnp.sum(parts) / inputs.size
    return loss, q


# speedup vs baseline: 1.0144x; 1.0144x over previous
import functools

import jax
import jax.numpy as jnp
from jax import lax
from jax.experimental import pallas as pl
from jax.experimental.pallas import tpu as pltpu, tpu_sc as plsc

N_HEADS = 4
N_EMB = 8192
HDIM = 16
N_TOK = 8192
TOK_BLK = 1024
N_WORKERS = 32
IDX_CHUNK = 128


def _tc_body(x_ref, wt_ref, idx_ref):
    idx_cols = []
    for h in range(N_HEADS):
        x_h = x_ref[:, h * HDIM:(h + 1) * HDIM]
        wt_h = wt_ref[h]
        s = lax.dot_general(x_h, wt_h, (((1,), (0,)), ((), ())),
                            preferred_element_type=jnp.float32)
        t = s - 0.5 * jnp.sum(wt_h * wt_h, axis=0)[None, :]
        i_h = jnp.argmax(t, axis=1).astype(jnp.int32)
        idx_cols.append(i_h + h * N_EMB)
    idx_ref[0] = jnp.stack(idx_cols, axis=1)


_B_PER_W = N_TOK * N_HEADS // N_WORKERS
_N_CHUNK = _B_PER_W // IDX_CHUNK


@functools.cache
def _make_sc_gather():
    @functools.partial(
        pl.kernel,
        out_type=(
            jax.ShapeDtypeStruct((N_TOK * N_HEADS, HDIM), jnp.float32),
            jax.ShapeDtypeStruct((N_WORKERS, HDIM), jnp.float32),
        ),
        mesh=plsc.VectorSubcoreMesh(core_axis_name="c", subcore_axis_name="s"),
        scratch_types=[
            pltpu.VMEM((_N_CHUNK, IDX_CHUNK), jnp.int32),
            pltpu.VMEM((_B_PER_W, HDIM), jnp.float32),
            pltpu.VMEM((_B_PER_W, HDIM), jnp.float32),
            pltpu.VMEM((HDIM,), jnp.float32),
            pltpu.SemaphoreType.DMA,
            pltpu.SemaphoreType.DMA,
        ],
        compiler_params=pltpu.CompilerParams(use_tc_tiling_on_sc=False),
    )
    def _sc_gather(idx_hbm, table_hbm, x_hbm, out_hbm, part_hbm,
                   idx_v, rows_v, x_v, part_v, sem, xsem):
        wid = lax.axis_index("s") * 2 + lax.axis_index("c")
        base = wid * _B_PER_W
        pltpu.sync_copy(idx_hbm.at[wid], idx_v)
        xcp = pltpu.async_copy(x_hbm.at[pl.ds(base, _B_PER_W)], x_v, xsem)
        copies = [
            pltpu.async_copy(table_hbm.at[idx_v.at[j]],
                             rows_v.at[pl.ds(j * IDX_CHUNK, IDX_CHUNK)], sem)
            for j in range(_N_CHUNK)
        ]
        for cp in copies:
            cp.wait()
        xcp.wait()

        def body(i, acc):
            d = rows_v[i] - x_v[i]
            return acc + d * d

        acc = lax.fori_loop(0, _B_PER_W, body, jnp.zeros((HDIM,), jnp.float32))
        part_v[...] = acc
        pltpu.sync_copy(part_v, part_hbm.at[wid])
        pltpu.sync_copy(rows_v, out_hbm.at[pl.ds(base, _B_PER_W)])

    return _sc_gather


def kernel(inputs, weights):
    x2d = inputs.reshape(N_TOK, N_HEADS * HDIM)
    wt = jnp.swapaxes(weights, 1, 2)

    n_tb = N_TOK // TOK_BLK
    gidx = pl.pallas_call(
        _tc_body,
        grid=(n_tb,),
        in_specs=[
            pl.BlockSpec((TOK_BLK, N_HEADS * HDIM), lambda t: (t, 0)),
            pl.BlockSpec((N_HEADS, HDIM, N_EMB), lambda t: (0, 0, 0)),
        ],
        out_specs=pl.BlockSpec((1, TOK_BLK, N_HEADS), lambda t: (t, 0, 0)),
        out_shape=jax.ShapeDtypeStruct((n_tb, TOK_BLK, N_HEADS), jnp.int32),
    )(x2d, wt)

    idx3 = gidx.reshape(N_WORKERS, _N_CHUNK, IDX_CHUNK)
    table = weights.reshape(N_HEADS * N_EMB, HDIM)
    xrows = inputs.reshape(N_TOK * N_HEADS, HDIM)
    q, parts = _make_sc_gather()(idx3, table, xrows)
    q = q.reshape(inputs.shape)

    loss = 0.25 * jnp.sum(parts) / inputs.size
    return loss, q
